# Initial kernel scaffold; baseline (speedup 1.0000x reference)
import jax
import jax.numpy as jnp
from jax.experimental import pallas as pl


def kernel(hidden_states, gate_w, expert_gate_w, expert_up_w, expert_down_w, shared_gate_w, shared_up_w, shared_down_w, shared_expert_gate_w):
    raise NotImplementedError("write your pallas kernel here")


# speedup vs baseline: 2.3621x; 2.3621x over previous
import jax
import jax.numpy as jnp
from jax.experimental import pallas as pl
from jax.experimental.pallas import tpu as pltpu

E = 8
TOP_K = 2
D = 1024
D_FF = 512
TB = 1024


def _silu(x):
    return x * jax.nn.sigmoid(x)


def _moe_kernel(x_ref, gate_w_ref, wg_ref, wu_ref, wd_ref,
                sg_ref, su_ref, sd_ref, segw_ref,
                out_ref, logits_ref,
                x16_ref, comb_ref):
    e = pl.program_id(1)

    @pl.when(e == 0)
    def _prologue():
        x = x_ref[...]
        logits = jax.lax.dot_general(
            x, gate_w_ref[...], (((1,), (1,)), ((), ())),
            precision=jax.lax.Precision.DEFAULT,
            preferred_element_type=jnp.float32)
        logits_ref[...] = logits
        m = jnp.max(logits, axis=1, keepdims=True)
        p = jnp.exp(logits - m)
        p = p / jnp.sum(p, axis=1, keepdims=True)
        iota = jax.lax.broadcasted_iota(jnp.int32, p.shape, 1)
        m1 = jnp.max(p, axis=1, keepdims=True)
        i1 = jnp.min(jnp.where(p == m1, iota, E), axis=1, keepdims=True)
        p2 = jnp.where(iota == i1, -jnp.inf, p)
        m2 = jnp.max(p2, axis=1, keepdims=True)
        i2 = jnp.min(jnp.where(p2 == m2, iota, E), axis=1, keepdims=True)
        comb_ref[...] = jnp.where(iota == i1, m1, 0.0) + jnp.where(iota == i2, m2, 0.0)

        x16 = x.astype(jnp.bfloat16)
        x16_ref[...] = x16

        sg16 = sg_ref[...].astype(jnp.bfloat16)
        su16 = su_ref[...].astype(jnp.bfloat16)
        sd16 = sd_ref[...].astype(jnp.bfloat16)
        g = jax.lax.dot_general(x16, sg16, (((1,), (1,)), ((), ())),
                                preferred_element_type=jnp.float32)
        u = jax.lax.dot_general(x16, su16, (((1,), (1,)), ((), ())),
                                preferred_element_type=jnp.float32)
        h = (_silu(g) * u).astype(jnp.bfloat16)
        ys = jax.lax.dot_general(h, sd16, (((1,), (1,)), ((), ())),
                                 preferred_element_type=jnp.float32)
        sgate_logit = jax.lax.dot_general(
            x, segw_ref[...], (((1,), (1,)), ((), ())),
            precision=jax.lax.Precision.HIGHEST,
            preferred_element_type=jnp.float32)
        out_ref[...] = jax.nn.sigmoid(sgate_logit) * ys

    x16 = x16_ref[...]
    wg16 = wg_ref[0].astype(jnp.bfloat16)
    wu16 = wu_ref[0].astype(jnp.bfloat16)
    wd16 = wd_ref[0].astype(jnp.bfloat16)
    g = jax.lax.dot_general(x16, wg16, (((1,), (1,)), ((), ())),
                            preferred_element_type=jnp.float32)
    u = jax.lax.dot_general(x16, wu16, (((1,), (1,)), ((), ())),
                            preferred_element_type=jnp.float32)
    comb = comb_ref[...]
    iota = jax.lax.broadcasted_iota(jnp.int32, comb.shape, 1)
    ce = jnp.sum(jnp.where(iota == e, comb, 0.0), axis=1, keepdims=True)
    h = (_silu(g) * u * ce).astype(jnp.bfloat16)
    y = jax.lax.dot_general(h, wd16, (((1,), (1,)), ((), ())),
                            preferred_element_type=jnp.float32)
    out_ref[...] += y


def kernel(hidden_states, gate_w, expert_gate_w, expert_up_w, expert_down_w,
           shared_gate_w, shared_up_w, shared_down_w, shared_expert_gate_w):
    b, s, d = hidden_states.shape
    x = hidden_states.reshape(-1, d)
    T = x.shape[0]
    n_tb = T // TB

    out, logits = pl.pallas_call(
        _moe_kernel,
        grid=(n_tb, E),
        in_specs=[
            pl.BlockSpec((TB, D), lambda t, e: (t, 0)),
            pl.BlockSpec((E, D), lambda t, e: (0, 0)),
            pl.BlockSpec((1, D_FF, D), lambda t, e: (e, 0, 0)),
            pl.BlockSpec((1, D_FF, D), lambda t, e: (e, 0, 0)),
            pl.BlockSpec((1, D, D_FF), lambda t, e: (e, 0, 0)),
            pl.BlockSpec((D_FF, D), lambda t, e: (0, 0)),
            pl.BlockSpec((D_FF, D), lambda t, e: (0, 0)),
            pl.BlockSpec((D, D_FF), lambda t, e: (0, 0)),
            pl.BlockSpec((1, D), lambda t, e: (0, 0)),
        ],
        out_specs=[
            pl.BlockSpec((TB, D), lambda t, e: (t, 0)),
            pl.BlockSpec((TB, E), lambda t, e: (t, 0)),
        ],
        out_shape=[
            jax.ShapeDtypeStruct((T, D), jnp.float32),
            jax.ShapeDtypeStruct((T, E), jnp.float32),
        ],
        scratch_shapes=[
            pltpu.VMEM((TB, D), jnp.bfloat16),
            pltpu.VMEM((TB, E), jnp.float32),
        ],
    )(x, gate_w, expert_gate_w, expert_up_w, expert_down_w,
      shared_gate_w, shared_up_w, shared_down_w, shared_expert_gate_w)

    return (out.reshape(b, s, d), logits)
